# baseline (speedup 1.0000x reference)
import dataclasses
import functools

import jax
import jax.numpy as jnp
from jax import lax
from jax.experimental import pallas as pl
from jax.experimental.pallas import tpu as pltpu
from jax.experimental.pallas import tpu_sc as plsc

B, N, K, W, D = 32, 256, 32, 100, 128
ALPHA = 0.2
E = N * K
H = K // 2
NH = N * H
CH = 128
NCHUNK = E // CH


def _lrelu(v):
    return jnp.where(v > 0, v, ALPHA * v)


def _prep_body(x_ref, wnT_ref, wn_ref, bn_ref, a2d_ref, idxe_ref, idxo_ref,
               blo_ref, bhi_ref, l_ref, attn_ref):
    x3 = x_ref[...]
    x2 = x3.reshape(B * N, W)
    wx = jnp.dot(x2, wnT_ref[...],
                 preferred_element_type=jnp.float32) + bn_ref[...]
    l_ref[...] = _lrelu(wx)

    waT = jnp.dot(a2d_ref[...], wn_ref[...],
                  preferred_element_type=jnp.float32)
    asum = a2d_ref[0:1, :] + a2d_ref[1:2, :]
    bsum = jnp.sum(bn_ref[...] * asum)
    s1 = jnp.sum(x3 * waT[0:1, :][None], axis=2)
    s2 = jnp.sum(x3 * waT[1:2, :][None], axis=2)
    s12 = jnp.concatenate([s1, s2], axis=1)

    jj = lax.broadcasted_iota(jnp.int32, (2 * N, NH), 0)
    q = ((jj == idxe_ref[...]) |
         (jj == (idxo_ref[...] + N))).astype(jnp.float32)
    e_hi = jnp.dot(s12, q, preferred_element_type=jnp.float32) + bsum
    e_lo = s1 + s2 + bsum

    shift = _lrelu(e_lo)
    lhi = _lrelu(e_hi) + bhi_ref[...]

    bc = (lax.broadcasted_iota(jnp.int32, (N, NH), 1) // H ==
          lax.broadcasted_iota(jnp.int32, (N, NH), 0)).astype(jnp.float32)
    sm = (lax.broadcasted_iota(jnp.int32, (NH, N), 0) // H ==
          lax.broadcasted_iota(jnp.int32, (NH, N), 1)).astype(jnp.float32)

    shift_w = jnp.dot(shift, bc, preferred_element_type=jnp.float32)
    p_hi = jnp.exp(lhi - shift_w)
    exp_blo = jnp.exp(blo_ref[...])
    sum_lo = jnp.dot(exp_blo, sm, preferred_element_type=jnp.float32)
    den = jnp.dot(p_hi, sm, preferred_element_type=jnp.float32) + sum_lo
    den_w = jnp.dot(den, bc, preferred_element_type=jnp.float32)
    alo_v = exp_blo / den_w
    ahi_v = p_hi / den_w
    attn_ref[...] = jnp.concatenate(
        [alo_v.reshape(B, N, H), ahi_v.reshape(B, N, H)], axis=2)


def _sc_body(l_hbm, idx_hbm, attn_hbm, out_hbm,
             ltile, idc0, idc1, atc0, atc1, obuf0, obuf1,
             osem0, osem1, isem0, isem1):
    b = lax.axis_index("c") * 16 + lax.axis_index("s")
    pltpu.sync_copy(l_hbm.at[b], ltile)

    att2 = attn_hbm.at[b]
    out2 = out_hbm.at[b]
    osems = (osem0, osem1)
    isems = (isem0, isem1)
    obufs = (obuf0, obuf1)
    idcs = (idc0, idc1)
    atcs = (atc0, atc1)

    def in_copies(c, half):
        return (
            pltpu.make_async_copy(idx_hbm.at[c], idcs[half].at[pl.ds(0, CH)],
                                  isems[half]),
            pltpu.make_async_copy(att2.at[c], atcs[half].at[pl.ds(0, CH)],
                                  isems[half]),
        )

    for half in range(2):
        for cp_ in in_copies(half, half):
            cp_.start()

    @pl.loop(0, NCHUNK, step=2)
    def _chunks(ci):
        for half in range(2):
            c = ci + half
            ob = obufs[half]

            for cp_ in in_copies(c, half):
                cp_.wait()

            @pl.when(c >= 2)
            def _():
                pltpu.make_async_copy(
                    ob, out2.at[pl.ds((c - 2) * CH, CH)], osems[half]).wait()

            lanes = lax.iota(jnp.int32, 16)
            idc = idcs[half]
            atc = atcs[half]

            @plsc.parallel_loop(0, CH, step=1, unroll=8)
            def _edges(e):
                rv = idc[pl.ds(e, 16)]
                av = atc[pl.ds(e, 16)]
                rsplat = jnp.broadcast_to(rv[0], (16,))
                asplat = jnp.broadcast_to(av[0], (16,))
                for cg in range(D // 16):
                    v = plsc.load_gather(ltile, [rsplat, lanes + cg * 16])
                    ob[e, pl.ds(cg * 16, 16)] = v * asplat

            @pl.when(c < NCHUNK - 2)
            def _():
                for cp_ in in_copies(c + 2, half):
                    cp_.start()

            pltpu.make_async_copy(
                ob, out2.at[pl.ds(c * CH, CH)], osems[half]).start()

    for half in range(2):
        pltpu.make_async_copy(
            obufs[half],
            out2.at[pl.ds((NCHUNK - 2 + half) * CH, CH)],
            osems[half]).wait()


def kernel(x_n, x_e, edge_indices, all_embeddings, lin_n_w, lin_n_b,
           lin_e_w, lin_e_b, a, bias_n, bias_e):
    idx2d = edge_indices[0].reshape(N, K)
    idx_e = idx2d[:, 0::2].reshape(1, NH)
    idx_o = idx2d[:, 1::2].reshape(1, NH)
    a2d = a.reshape(2, D)
    wnT = lin_n_w.T
    bn = lin_n_b.reshape(1, D)
    blo = bias_n[:, :H].reshape(1, NH)
    bhi = bias_n[:, H:].reshape(1, NH)

    l_flat, attn = pl.pallas_call(
        _prep_body,
        grid=(1,),
        in_specs=[
            pl.BlockSpec((B, N, W), lambda i: (0, 0, 0)),
            pl.BlockSpec((W, D), lambda i: (0, 0)),
            pl.BlockSpec((D, W), lambda i: (0, 0)),
            pl.BlockSpec((1, D), lambda i: (0, 0)),
            pl.BlockSpec((2, D), lambda i: (0, 0)),
            pl.BlockSpec((1, NH), lambda i: (0, 0)),
            pl.BlockSpec((1, NH), lambda i: (0, 0)),
            pl.BlockSpec((1, NH), lambda i: (0, 0)),
            pl.BlockSpec((1, NH), lambda i: (0, 0)),
        ],
        out_specs=[
            pl.BlockSpec((B * N, D), lambda i: (0, 0)),
            pl.BlockSpec((B, N, K), lambda i: (0, 0, 0)),
        ],
        out_shape=[
            jax.ShapeDtypeStruct((B * N, D), jnp.float32),
            jax.ShapeDtypeStruct((B, N, K), jnp.float32),
        ],
    )(x_n, wnT, lin_n_w, bn, a2d, idx_e, idx_o, blo, bhi)

    l3 = l_flat.reshape(B, N, D)
    attn3 = attn.reshape(B, NCHUNK, CH)
    idx2 = edge_indices[0].reshape(NCHUNK, CH)

    cp = pltpu.CompilerParams()
    fields = pltpu.CompilerParams.__dataclass_fields__
    if "needs_layout_passes" in fields:
        cp = dataclasses.replace(cp, needs_layout_passes=False)
    if "use_tc_tiling_on_sc" in fields:
        cp = dataclasses.replace(cp, use_tc_tiling_on_sc=True)
    sc_kernel = functools.partial(
        pl.kernel,
        out_type=jax.ShapeDtypeStruct((B, E, D), jnp.float32),
        mesh=plsc.VectorSubcoreMesh(core_axis_name="c", subcore_axis_name="s"),
        compiler_params=cp,
        scratch_types=[
            pltpu.VMEM((N, D), jnp.float32),
            pltpu.VMEM((CH + 16,), jnp.int32),
            pltpu.VMEM((CH + 16,), jnp.int32),
            pltpu.VMEM((CH + 16,), jnp.float32),
            pltpu.VMEM((CH + 16,), jnp.float32),
            pltpu.VMEM((CH, D), jnp.float32),
            pltpu.VMEM((CH, D), jnp.float32),
            pltpu.SemaphoreType.DMA,
            pltpu.SemaphoreType.DMA,
            pltpu.SemaphoreType.DMA,
            pltpu.SemaphoreType.DMA,
        ],
    )(_sc_body)

    out = sc_kernel(l3, idx2, attn3)
    return out.reshape(B, N, K, D)


# speedup vs baseline: 1.1543x; 1.1543x over previous
import dataclasses
import functools

import jax
import jax.numpy as jnp
from jax import lax
from jax.experimental import pallas as pl
from jax.experimental.pallas import tpu as pltpu
from jax.experimental.pallas import tpu_sc as plsc

B, N, K, W, D = 32, 256, 32, 100, 128
ALPHA = 0.2
E = N * K
H = K // 2
NH = N * H
CH = 256
NCHUNK = E // CH


def _lrelu(v):
    return jnp.where(v > 0, v, ALPHA * v)


def _prep_body(x_ref, wnT_ref, wn_ref, bn_ref, a2d_ref, idxe_ref, idxo_ref,
               blo_ref, bhi_ref, l_ref, attn_ref):
    x3 = x_ref[...]
    x2 = x3.reshape(B * N, W)
    wx = jnp.dot(x2, wnT_ref[...],
                 preferred_element_type=jnp.float32) + bn_ref[...]
    l_ref[...] = _lrelu(wx)

    waT = jnp.dot(a2d_ref[...], wn_ref[...],
                  preferred_element_type=jnp.float32)
    asum = a2d_ref[0:1, :] + a2d_ref[1:2, :]
    bsum = jnp.sum(bn_ref[...] * asum)
    s1 = jnp.sum(x3 * waT[0:1, :][None], axis=2)
    s2 = jnp.sum(x3 * waT[1:2, :][None], axis=2)
    s12 = jnp.concatenate([s1, s2], axis=1)

    jj = lax.broadcasted_iota(jnp.int32, (2 * N, NH), 0)
    q = ((jj == idxe_ref[...]) |
         (jj == (idxo_ref[...] + N))).astype(jnp.float32)
    e_hi = jnp.dot(s12, q, preferred_element_type=jnp.float32) + bsum
    e_lo = s1 + s2 + bsum

    shift = _lrelu(e_lo)
    lhi = _lrelu(e_hi) + bhi_ref[...]

    bc = (lax.broadcasted_iota(jnp.int32, (N, NH), 1) // H ==
          lax.broadcasted_iota(jnp.int32, (N, NH), 0)).astype(jnp.float32)
    sm = (lax.broadcasted_iota(jnp.int32, (NH, N), 0) // H ==
          lax.broadcasted_iota(jnp.int32, (NH, N), 1)).astype(jnp.float32)

    shift_w = jnp.dot(shift, bc, preferred_element_type=jnp.float32)
    p_hi = jnp.exp(lhi - shift_w)
    exp_blo = jnp.exp(blo_ref[...])
    sum_lo = jnp.dot(exp_blo, sm, preferred_element_type=jnp.float32)
    den = jnp.dot(p_hi, sm, preferred_element_type=jnp.float32) + sum_lo
    den_w = jnp.dot(den, bc, preferred_element_type=jnp.float32)
    alo_v = exp_blo / den_w
    ahi_v = p_hi / den_w
    attn_ref[...] = jnp.concatenate(
        [alo_v.reshape(B, N, H), ahi_v.reshape(B, N, H)], axis=2)


def _sc_body(l_hbm, idx_hbm, attn_hbm, out_hbm,
             ltile, idxv, attv, obuf0, obuf1, osem0, osem1):
    b = lax.axis_index("c") * 16 + lax.axis_index("s")
    pltpu.sync_copy(l_hbm.at[b], ltile)
    pltpu.sync_copy(idx_hbm, idxv.at[pl.ds(0, E)])
    pltpu.sync_copy(attn_hbm.at[b], attv.at[pl.ds(0, E)])

    out2 = out_hbm.at[b]
    osems = (osem0, osem1)
    obufs = (obuf0, obuf1)

    @pl.loop(0, NCHUNK, step=2)
    def _chunks(ci):
        for half in range(2):
            c = ci + half
            ob = obufs[half]

            @pl.when(c >= 2)
            def _():
                pltpu.make_async_copy(
                    ob, out2.at[pl.ds((c - 2) * CH, CH)], osems[half]).wait()

            base = c * CH
            lanes = lax.iota(jnp.int32, 16)

            @plsc.parallel_loop(0, CH, step=1, unroll=8)
            def _edges(e):
                rv = idxv[pl.ds(base + e, 16)]
                av = attv[pl.ds(base + e, 16)]
                rsplat = jnp.broadcast_to(rv[0], (16,))
                asplat = jnp.broadcast_to(av[0], (16,))
                for cg in range(D // 16):
                    v = plsc.load_gather(ltile, [rsplat, lanes + cg * 16])
                    ob[e, pl.ds(cg * 16, 16)] = v * asplat

            pltpu.make_async_copy(
                ob, out2.at[pl.ds(c * CH, CH)], osems[half]).start()

    for half in range(2):
        pltpu.make_async_copy(
            obufs[half],
            out2.at[pl.ds((NCHUNK - 2 + half) * CH, CH)],
            osems[half]).wait()


def kernel(x_n, x_e, edge_indices, all_embeddings, lin_n_w, lin_n_b,
           lin_e_w, lin_e_b, a, bias_n, bias_e):
    idx2d = edge_indices[0].reshape(N, K)
    idx_e = idx2d[:, 0::2].reshape(1, NH)
    idx_o = idx2d[:, 1::2].reshape(1, NH)
    a2d = a.reshape(2, D)
    wnT = lin_n_w.T
    bn = lin_n_b.reshape(1, D)
    blo = bias_n[:, :H].reshape(1, NH)
    bhi = bias_n[:, H:].reshape(1, NH)

    l_flat, attn = pl.pallas_call(
        _prep_body,
        grid=(1,),
        in_specs=[
            pl.BlockSpec((B, N, W), lambda i: (0, 0, 0)),
            pl.BlockSpec((W, D), lambda i: (0, 0)),
            pl.BlockSpec((D, W), lambda i: (0, 0)),
            pl.BlockSpec((1, D), lambda i: (0, 0)),
            pl.BlockSpec((2, D), lambda i: (0, 0)),
            pl.BlockSpec((1, NH), lambda i: (0, 0)),
            pl.BlockSpec((1, NH), lambda i: (0, 0)),
            pl.BlockSpec((1, NH), lambda i: (0, 0)),
            pl.BlockSpec((1, NH), lambda i: (0, 0)),
        ],
        out_specs=[
            pl.BlockSpec((B * N, D), lambda i: (0, 0)),
            pl.BlockSpec((B, N, K), lambda i: (0, 0, 0)),
        ],
        out_shape=[
            jax.ShapeDtypeStruct((B * N, D), jnp.float32),
            jax.ShapeDtypeStruct((B, N, K), jnp.float32),
        ],
    )(x_n, wnT, lin_n_w, bn, a2d, idx_e, idx_o, blo, bhi)

    l3 = l_flat.reshape(B, N, D)
    attn2 = attn.reshape(B, E)
    idx_flat = edge_indices[0].reshape(E)

    cp = pltpu.CompilerParams()
    if "needs_layout_passes" in pltpu.CompilerParams.__dataclass_fields__:
        cp = dataclasses.replace(cp, needs_layout_passes=False)
    sc_kernel = functools.partial(
        pl.kernel,
        out_type=jax.ShapeDtypeStruct((B, E, D), jnp.float32),
        mesh=plsc.VectorSubcoreMesh(core_axis_name="c", subcore_axis_name="s"),
        compiler_params=cp,
        scratch_types=[
            pltpu.VMEM((N, D), jnp.float32),
            pltpu.VMEM((E + 16,), jnp.int32),
            pltpu.VMEM((E + 16,), jnp.float32),
            pltpu.VMEM((CH, D), jnp.float32),
            pltpu.VMEM((CH, D), jnp.float32),
            pltpu.SemaphoreType.DMA,
            pltpu.SemaphoreType.DMA,
        ],
    )(_sc_body)

    out = sc_kernel(l3, idx_flat, attn2)
    return out.reshape(B, N, K, D)
